# baseline (speedup 1.0000x reference)
import jax
import jax.numpy as jnp
from jax.experimental import pallas as pl
from jax.experimental.pallas import tpu as pltpu

_TOP_K = 8
_N_EXPERTS = 64


def _gate_kernel(x_ref, w_ref, idx_ref, wgt_ref, acc_ref, *, kc, nk):
    k = pl.program_id(1)
    wk = w_ref[:, pl.ds(k * kc, kc)]
    part = jax.lax.dot_general(
        wk, x_ref[...], dimension_numbers=(((1,), (1,)), ((), ())),
        preferred_element_type=jnp.float32)

    @pl.when(k == 0)
    def _():
        acc_ref[...] = part

    @pl.when(k > 0)
    def _():
        acc_ref[...] = acc_ref[...] + part

    @pl.when(k == nk - 1)
    def _():
        logits = acc_ref[...]
        row = jax.lax.broadcasted_iota(jnp.int32, logits.shape, 0)
        vals = logits
        sel_v = []
        sel_i = []
        for _ in range(_TOP_K):
            m = jnp.max(vals, axis=0, keepdims=True)
            cand = jnp.where(vals == m, row, _N_EXPERTS)
            a = jnp.min(cand, axis=0, keepdims=True)
            sel_v.append(m)
            sel_i.append(a)
            vals = jnp.where(row == a, -jnp.inf, vals)
        top_v = jnp.concatenate(sel_v, axis=0)
        top_i = jnp.concatenate(sel_i, axis=0)
        e = jnp.exp(top_v - top_v[0:1, :])
        wgt = e / jnp.sum(e, axis=0, keepdims=True)
        idx_ref[...] = top_i
        wgt_ref[...] = wgt


def kernel(hidden_states, weight):
    bsz, seq_len, dim = hidden_states.shape
    n_tokens = bsz * seq_len
    x = hidden_states.reshape(n_tokens, dim)
    block = 512
    kc = 1024
    nk = dim // kc
    import functools
    body = functools.partial(_gate_kernel, kc=kc, nk=nk)
    idx_t, wgt_t = pl.pallas_call(
        body,
        grid=(n_tokens // block, nk),
        compiler_params=pltpu.CompilerParams(
            dimension_semantics=("parallel", "arbitrary")),
        in_specs=[
            pl.BlockSpec((block, kc), lambda i, k: (i, k)),
            pl.BlockSpec((_N_EXPERTS, dim), lambda i, k: (0, 0)),
        ],
        out_specs=[
            pl.BlockSpec((_TOP_K, block), lambda i, k: (0, i)),
            pl.BlockSpec((_TOP_K, block), lambda i, k: (0, i)),
        ],
        out_shape=[
            jax.ShapeDtypeStruct((_TOP_K, n_tokens), jnp.int32),
            jax.ShapeDtypeStruct((_TOP_K, n_tokens), jnp.float32),
        ],
        scratch_shapes=[pltpu.VMEM((_N_EXPERTS, block), jnp.float32)],
    )(x, weight)
    aux_loss = jnp.asarray(0.0, dtype=hidden_states.dtype)
    return idx_t.T, wgt_t.T.astype(hidden_states.dtype), aux_loss


# speedup vs baseline: 1.5330x; 1.5330x over previous
import jax
import jax.numpy as jnp
from jax.experimental import pallas as pl
from jax.experimental.pallas import tpu as pltpu

_TOP_K = 8
_N_EXPERTS = 64


def _gate_kernel(x_ref, w_ref, idx_ref, wgt_ref):
    x = x_ref[...]
    w = w_ref[...]
    logits = jax.lax.dot_general(
        w, x, dimension_numbers=(((1,), (1,)), ((), ())),
        preferred_element_type=jnp.float32)
    row = jax.lax.broadcasted_iota(jnp.int32, logits.shape, 0)
    vals = logits
    sel_v = []
    sel_i = []
    for _ in range(_TOP_K):
        m = jnp.max(vals, axis=0, keepdims=True)
        cand = jnp.where(vals == m, row, _N_EXPERTS)
        a = jnp.min(cand, axis=0, keepdims=True)
        sel_v.append(m)
        sel_i.append(a)
        vals = jnp.where(row == a, -jnp.inf, vals)
    top_v = jnp.concatenate(sel_v, axis=0)
    top_i = jnp.concatenate(sel_i, axis=0)
    e = jnp.exp(top_v - top_v[0:1, :])
    wgt = e / jnp.sum(e, axis=0, keepdims=True)
    idx_ref[...] = top_i
    wgt_ref[...] = wgt


def kernel(hidden_states, weight):
    bsz, seq_len, dim = hidden_states.shape
    n_tokens = bsz * seq_len
    x = hidden_states.reshape(n_tokens, dim)
    block = 1024
    idx_t, wgt_t = pl.pallas_call(
        _gate_kernel,
        grid=(n_tokens // block,),
        compiler_params=pltpu.CompilerParams(
            dimension_semantics=("parallel",)),
        in_specs=[
            pl.BlockSpec((block, dim), lambda i: (i, 0)),
            pl.BlockSpec((_N_EXPERTS, dim), lambda i: (0, 0)),
        ],
        out_specs=[
            pl.BlockSpec((_TOP_K, block), lambda i: (0, i)),
            pl.BlockSpec((_TOP_K, block), lambda i: (0, i)),
        ],
        out_shape=[
            jax.ShapeDtypeStruct((_TOP_K, n_tokens), jnp.int32),
            jax.ShapeDtypeStruct((_TOP_K, n_tokens), jnp.float32),
        ],
    )(x, weight)
    aux_loss = jnp.asarray(0.0, dtype=hidden_states.dtype)
    return idx_t.T, wgt_t.T.astype(hidden_states.dtype), aux_loss


# speedup vs baseline: 1.5924x; 1.0387x over previous
import jax
import jax.numpy as jnp
from jax.experimental import pallas as pl
from jax.experimental.pallas import tpu as pltpu

_TOP_K = 8
_N_EXPERTS = 64


_KH = 2048


def _gate_kernel(x_ref, w_ref, idx_ref, wgt_ref):
    xa = x_ref[:, :_KH]
    xb = x_ref[:, _KH:]
    wa = w_ref[:, :_KH]
    wb = w_ref[:, _KH:]
    la = jax.lax.dot_general(
        xa, wa, dimension_numbers=(((1,), (1,)), ((), ())),
        preferred_element_type=jnp.float32)
    lb = jax.lax.dot_general(
        wb, xb, dimension_numbers=(((1,), (1,)), ((), ())),
        preferred_element_type=jnp.float32)
    logits = lb + la.T
    row = jax.lax.broadcasted_iota(jnp.int32, logits.shape, 0)
    vals = logits
    sel_v = []
    sel_i = []
    for _ in range(_TOP_K):
        m = jnp.max(vals, axis=0, keepdims=True)
        cand = jnp.where(vals == m, row, _N_EXPERTS)
        a = jnp.min(cand, axis=0, keepdims=True)
        sel_v.append(m)
        sel_i.append(a)
        vals = jnp.where(row == a, -jnp.inf, vals)
    top_v = jnp.concatenate(sel_v, axis=0)
    top_i = jnp.concatenate(sel_i, axis=0)
    e = jnp.exp(top_v - top_v[0:1, :])
    wgt = e / jnp.sum(e, axis=0, keepdims=True)
    idx_ref[...] = top_i
    wgt_ref[...] = wgt


def kernel(hidden_states, weight):
    bsz, seq_len, dim = hidden_states.shape
    n_tokens = bsz * seq_len
    x = hidden_states.reshape(n_tokens, dim)
    block = 512
    idx_t, wgt_t = pl.pallas_call(
        _gate_kernel,
        grid=(n_tokens // block,),
        compiler_params=pltpu.CompilerParams(
            dimension_semantics=("parallel",)),
        in_specs=[
            pl.BlockSpec((block, dim), lambda i: (i, 0)),
            pl.BlockSpec((_N_EXPERTS, dim), lambda i: (0, 0)),
        ],
        out_specs=[
            pl.BlockSpec((_TOP_K, block), lambda i: (0, i)),
            pl.BlockSpec((_TOP_K, block), lambda i: (0, i)),
        ],
        out_shape=[
            jax.ShapeDtypeStruct((_TOP_K, n_tokens), jnp.int32),
            jax.ShapeDtypeStruct((_TOP_K, n_tokens), jnp.float32),
        ],
    )(x, weight)
    aux_loss = jnp.asarray(0.0, dtype=hidden_states.dtype)
    return idx_t.T, wgt_t.T.astype(hidden_states.dtype), aux_loss
